# baseline (speedup 1.0000x reference)
import functools

import jax
import jax.numpy as jnp
from jax import lax
from jax.experimental import pallas as pl
from jax.experimental.pallas import tpu as pltpu
from jax.experimental.pallas import tpu_sc as plsc

D = 64
B_TOTAL = 16384 * 26
NC, NS = 2, 16
NW = NC * NS
B_PER_W = B_TOTAL // NW
CHUNK = 832
N_CHUNKS = B_PER_W // CHUNK

_mesh = plsc.VectorSubcoreMesh(core_axis_name="c", subcore_axis_name="s")


@functools.partial(
    pl.kernel,
    mesh=_mesh,
    out_type=jax.ShapeDtypeStruct((B_TOTAL, D), jnp.float32),
    scratch_types=[
        pltpu.VMEM((CHUNK,), jnp.int32),
        pltpu.VMEM((CHUNK, D), jnp.float32),
        pltpu.SemaphoreType.DMA,
    ],
    compiler_params=pltpu.CompilerParams(use_tc_tiling_on_sc=False),
)
def _gather_sc(table_hbm, idx_hbm, out_hbm, idx_v, rows_v, sem):
    wid = lax.axis_index("s") * NC + lax.axis_index("c")
    base = wid * B_PER_W

    def body(i, carry):
        off = base + i * CHUNK
        pltpu.sync_copy(idx_hbm.at[pl.ds(off, CHUNK)], idx_v)
        pltpu.async_copy(table_hbm.at[idx_v], rows_v, sem).wait()
        pltpu.sync_copy(rows_v, out_hbm.at[pl.ds(off, CHUNK)])
        return carry

    lax.fori_loop(0, N_CHUNKS, body, 0)


def kernel(embedding, indices):
    idx = indices.reshape(-1).astype(jnp.int32)
    out = _gather_sc(embedding, idx)
    return out.reshape(indices.shape + (embedding.shape[1],))


# speedup vs baseline: 1.0093x; 1.0093x over previous
import functools

import jax
import jax.numpy as jnp
from jax import lax
from jax.experimental import pallas as pl
from jax.experimental.pallas import tpu as pltpu
from jax.experimental.pallas import tpu_sc as plsc

D = 64
B_TOTAL = 16384 * 26
NC, NS = 2, 16
NW = NC * NS
B_PER_W = B_TOTAL // NW
NBUF = 4
CHUNK = 416
N_CHUNKS = B_PER_W // CHUNK
N_OUTER = N_CHUNKS // NBUF

_mesh = plsc.VectorSubcoreMesh(core_axis_name="c", subcore_axis_name="s")


@functools.partial(
    pl.kernel,
    mesh=_mesh,
    out_type=jax.ShapeDtypeStruct((B_TOTAL, D), jnp.float32),
    scratch_types=[
        pltpu.VMEM((B_PER_W,), jnp.int32),
        pltpu.VMEM((NBUF, CHUNK, D), jnp.float32),
        pltpu.SemaphoreType.DMA((NBUF,)),
        pltpu.SemaphoreType.DMA((NBUF,)),
    ],
    compiler_params=pltpu.CompilerParams(use_tc_tiling_on_sc=False),
)
def _gather_sc(table_hbm, idx_hbm, out_hbm, idx_v, rows_v, gsem, psem):
    wid = lax.axis_index("s") * NC + lax.axis_index("c")
    base = wid * B_PER_W
    pltpu.sync_copy(idx_hbm.at[pl.ds(base, B_PER_W)], idx_v)

    def start_gather(i, b):
        pltpu.async_copy(
            table_hbm.at[idx_v.at[pl.ds(i * CHUNK, CHUNK)]],
            rows_v.at[b],
            gsem.at[b],
        )

    def start_put(i, b):
        pltpu.async_copy(
            rows_v.at[b],
            out_hbm.at[pl.ds(base + i * CHUNK, CHUNK)],
            psem.at[b],
        )

    def wait_gather(i, b):
        pltpu.make_async_copy(
            table_hbm.at[idx_v.at[pl.ds(i * CHUNK, CHUNK)]],
            rows_v.at[b],
            gsem.at[b],
        ).wait()

    def wait_put(i, b):
        pltpu.make_async_copy(
            rows_v.at[b],
            out_hbm.at[pl.ds(base + i * CHUNK, CHUNK)],
            psem.at[b],
        ).wait()

    for b in range(NBUF):
        start_gather(b, b)

    def outer(k, carry):
        g = k * NBUF
        for b in range(NBUF):
            wait_gather(g + b, b)
            start_put(g + b, b)
        for b in range(NBUF):
            wait_put(g + b, b)
            start_gather(g + b + NBUF, b)
        return carry

    lax.fori_loop(0, N_OUTER - 1, outer, 0)

    g = (N_OUTER - 1) * NBUF
    for b in range(NBUF):
        wait_gather(g + b, b)
        start_put(g + b, b)
    for b in range(NBUF):
        wait_put(g + b, b)


def kernel(embedding, indices):
    idx = indices.reshape(-1).astype(jnp.int32)
    out = _gather_sc(embedding, idx)
    return out.reshape(indices.shape + (embedding.shape[1],))


# speedup vs baseline: 1.0102x; 1.0009x over previous
import functools

import jax
import jax.numpy as jnp
from jax import lax
from jax.experimental import pallas as pl
from jax.experimental.pallas import tpu as pltpu
from jax.experimental.pallas import tpu_sc as plsc

D = 64
BATCH = 16384
FIELDS = 26
NC, NS = 2, 16
NW = NC * NS
B_PER_W = BATCH // NW
CHUNKB = 32
N_CHUNKS = B_PER_W // CHUNKB

_mesh = plsc.VectorSubcoreMesh(core_axis_name="c", subcore_axis_name="s")


@functools.partial(
    pl.kernel,
    mesh=_mesh,
    out_type=jax.ShapeDtypeStruct((BATCH, FIELDS, D), jnp.float32),
    scratch_types=[
        pltpu.VMEM((B_PER_W, FIELDS), jnp.int32),
        pltpu.VMEM((2, CHUNKB, FIELDS, D), jnp.float32),
        pltpu.SemaphoreType.DMA((2,)),
        pltpu.SemaphoreType.DMA((2,)),
    ],
    compiler_params=pltpu.CompilerParams(use_tc_tiling_on_sc=False),
)
def _gather_sc(table_hbm, idx_hbm, out_hbm, idx_v, rows_v, gsem, psem):
    wid = lax.axis_index("s") * NC + lax.axis_index("c")
    b0 = wid * B_PER_W
    pltpu.sync_copy(idx_hbm.at[pl.ds(b0, B_PER_W)], idx_v)

    def start_gathers(k, buf):
        for j in range(CHUNKB):
            pltpu.async_copy(
                table_hbm.at[idx_v.at[k * CHUNKB + j]],
                rows_v.at[buf, j],
                gsem.at[buf],
            )

    def wait_gathers(k, buf):
        for j in range(CHUNKB):
            pltpu.make_async_copy(
                table_hbm.at[idx_v.at[k * CHUNKB + j]],
                rows_v.at[buf, j],
                gsem.at[buf],
            ).wait()

    def start_put(k, buf):
        pltpu.async_copy(
            rows_v.at[buf],
            out_hbm.at[pl.ds(b0 + k * CHUNKB, CHUNKB)],
            psem.at[buf],
        )

    def wait_put(k, buf):
        pltpu.make_async_copy(
            rows_v.at[buf],
            out_hbm.at[pl.ds(b0 + k * CHUNKB, CHUNKB)],
            psem.at[buf],
        ).wait()

    for b in range(2):
        start_gathers(b, b)

    def outer(i, carry):
        k = i * 2
        for b in range(2):
            wait_gathers(k + b, b)
            start_put(k + b, b)
        for b in range(2):
            wait_put(k + b, b)
            start_gathers(k + b + 2, b)
        return carry

    lax.fori_loop(0, N_CHUNKS // 2 - 1, outer, 0)

    k = N_CHUNKS - 2
    for b in range(2):
        wait_gathers(k + b, b)
        start_put(k + b, b)
    for b in range(2):
        wait_put(k + b, b)


def kernel(embedding, indices):
    return _gather_sc(embedding, indices.astype(jnp.int32))


# speedup vs baseline: 1.0116x; 1.0014x over previous
import functools

import jax
import jax.numpy as jnp
from jax import lax
from jax.experimental import pallas as pl
from jax.experimental.pallas import tpu as pltpu
from jax.experimental.pallas import tpu_sc as plsc

D = 64
DP = 128
BATCH = 16384
FIELDS = 26
VOCAB = 1000000
NC, NS = 2, 16
NW = NC * NS
B_PER_W = BATCH // NW
CB = 128
CHUNKS = B_PER_W // CB

_mesh = plsc.VectorSubcoreMesh(core_axis_name="c", subcore_axis_name="s")


@functools.partial(
    pl.kernel,
    mesh=_mesh,
    out_type=jax.ShapeDtypeStruct((FIELDS, BATCH, DP), jnp.float32),
    scratch_types=[
        pltpu.VMEM((FIELDS * B_PER_W,), jnp.int32),
        pltpu.VMEM((2, CB, DP), jnp.float32),
        pltpu.SemaphoreType.DMA((2,)),
        pltpu.SemaphoreType.DMA((2,)),
    ],
    compiler_params=pltpu.CompilerParams(use_tc_tiling_on_sc=True),
)
def _gather_sc(table_hbm, idx_hbm, out_hbm, idx_v, rows_v, gsem, psem):
    wid = lax.axis_index("s") * NC + lax.axis_index("c")
    b0 = wid * B_PER_W

    def stage_idx(f, carry):
        pltpu.sync_copy(
            idx_hbm.at[f, pl.ds(b0, B_PER_W)],
            idx_v.at[pl.ds(f * B_PER_W, B_PER_W)],
        )
        return carry

    lax.fori_loop(0, FIELDS, stage_idx, 0)

    def start_gather(f, h, buf):
        pltpu.async_copy(
            table_hbm.at[idx_v.at[pl.ds(f * B_PER_W + h * CB, CB)]],
            rows_v.at[buf],
            gsem.at[buf],
        )

    def wait_gather(f, h, buf):
        pltpu.make_async_copy(
            table_hbm.at[idx_v.at[pl.ds(f * B_PER_W + h * CB, CB)]],
            rows_v.at[buf],
            gsem.at[buf],
        ).wait()

    def start_put(f, h, buf):
        pltpu.async_copy(
            rows_v.at[buf],
            out_hbm.at[f, pl.ds(b0 + h * CB, CB)],
            psem.at[buf],
        )

    def wait_put(f, h, buf):
        pltpu.make_async_copy(
            rows_v.at[buf],
            out_hbm.at[f, pl.ds(b0 + h * CB, CB)],
            psem.at[buf],
        ).wait()

    def do_step(s, carry):
        f = s // CHUNKS
        h = s % CHUNKS
        buf = s % 2

        @pl.when(s >= 2)
        def _():
            wait_put(f, h, buf)

        start_gather(f, h, buf)
        wait_gather(f, h, buf)
        start_put(f, h, buf)
        return carry

    lax.fori_loop(0, FIELDS * CHUNKS, do_step, 0)

    for buf in range(2):
        wait_put(FIELDS - 1, CHUNKS - 2 + buf, buf)


def kernel(embedding, indices):
    tpad = jnp.pad(embedding, ((0, 0), (0, DP - D)))
    out = _gather_sc(tpad, indices.T.astype(jnp.int32))
    return out[:, :, :D].transpose(1, 0, 2)
